# baseline (speedup 1.0000x reference)
import jax
import jax.numpy as jnp
from jax import lax
from jax.experimental import pallas as pl
from jax.experimental.pallas import tpu as pltpu
from jax.experimental.pallas import tpu_sc as plsc

DIM = 32
SUB = 8
BATCH = 16384
NC, NS, L = 2, 16, 16
NW = NC * NS
BPW = BATCH // NW
CH = 32
NCH = BPW // CH


def _sc_body(aidx_hbm, bidx_hbm, a_hbm, b_hbm, out_hbm,
             aidx_v, bidx_v, atile_v, btile_v, abuf, bbuf, out_v, sema, semb):
    wid = lax.axis_index("s") * NC + lax.axis_index("c")
    base = wid * BPW

    pltpu.sync_copy(aidx_hbm.at[pl.ds(base, BPW)], aidx_v)
    pltpu.sync_copy(bidx_hbm.at[pl.ds(base, BPW)], bidx_v)

    iota = lax.iota(jnp.int32, L)

    def scale(v, carry):
        off = pl.multiple_of(v * L, L)
        sl = pl.ds(off, L)
        atile_v[sl] = lax.shift_right_logical(aidx_v[sl], 3)
        btile_v[sl] = lax.shift_right_logical(bidx_v[sl], 3)
        return carry

    lax.fori_loop(0, BPW // L, scale, 0)

    def chunk(k, carry):
        coff = pl.multiple_of(k * CH, CH)
        csl = pl.ds(coff, CH)
        ca = pltpu.async_copy(a_hbm.at[atile_v.at[csl]], abuf, sema)
        cb = pltpu.async_copy(b_hbm.at[btile_v.at[csl]], bbuf, semb)
        ca.wait()
        cb.wait()
        raws = []
        for g in range(CH // L):
            sl = pl.ds(coff + g * L, L)
            raws.append((aidx_v[sl], bidx_v[sl]))
        for g, (araw, braw) in enumerate(raws):
            acc = jnp.zeros((L,), jnp.float32)
            for j in range(L):
                i = g * L + j
                sa = lax.bitwise_and(araw[j], 7)
                sb = lax.bitwise_and(braw[j], 7)
                p = (abuf[i, sa, pl.ds(0, L)] * bbuf[i, sb, pl.ds(0, L)]
                     + abuf[i, sa, pl.ds(L, L)] * bbuf[i, sb, pl.ds(L, L)])
                acc = jnp.where(iota == j, jnp.sum(p), acc)
            out_v[pl.ds(coff + g * L, L)] = acc
        return carry

    lax.fori_loop(0, NCH, chunk, 0)

    pltpu.sync_copy(out_v, out_hbm.at[pl.ds(base, BPW)])


def kernel(aIdx, bIdx, A, B):
    num = A.shape[0]
    k = pl.kernel(
        _sc_body,
        out_type=jax.ShapeDtypeStruct((BATCH,), jnp.float32),
        mesh=plsc.VectorSubcoreMesh(core_axis_name="c", subcore_axis_name="s"),
        compiler_params=pltpu.CompilerParams(
            needs_layout_passes=False, use_tc_tiling_on_sc=False),
        scratch_types=[
            pltpu.VMEM((BPW,), jnp.int32),
            pltpu.VMEM((BPW,), jnp.int32),
            pltpu.VMEM((BPW,), jnp.int32),
            pltpu.VMEM((BPW,), jnp.int32),
            pltpu.VMEM((CH, SUB, DIM), jnp.float32),
            pltpu.VMEM((CH, SUB, DIM), jnp.float32),
            pltpu.VMEM((BPW,), jnp.float32),
            pltpu.SemaphoreType.DMA,
            pltpu.SemaphoreType.DMA,
        ],
    )
    a3 = A.reshape(num // SUB, SUB, DIM)
    b3 = B.reshape(num // SUB, SUB, DIM)
    return k(aIdx.astype(jnp.int32), bIdx.astype(jnp.int32), a3, b3)


# speedup vs baseline: 2.3190x; 2.3190x over previous
import jax
import jax.numpy as jnp
from jax import lax
from jax.experimental import pallas as pl
from jax.experimental.pallas import tpu as pltpu
from jax.experimental.pallas import tpu_sc as plsc

DIM = 32
SUB = 8
BATCH = 16384
NC, NS, L = 2, 16, 16
NW = NC * NS
BPW = BATCH // NW
CH = 16
NCH = BPW // CH


def _sc_body(aidx_hbm, bidx_hbm, a_hbm, b_hbm, out_hbm,
             aidx_v, bidx_v, abuf, bbuf, abuf1, bbuf1, out_v, sema, semb):
    wid = lax.axis_index("s") * NC + lax.axis_index("c")
    base = wid * BPW

    pltpu.sync_copy(aidx_hbm.at[pl.ds(base, BPW)], aidx_v)
    pltpu.sync_copy(bidx_hbm.at[pl.ds(base, BPW)], bidx_v)

    iota = lax.iota(jnp.int32, L)

    def half(m, h, abuf, bbuf):
        coff = pl.multiple_of((2 * m + h) * CH, CH)
        copies = []
        raws = []
        for g in range(CH // L):
            sl = pl.ds(coff + g * L, L)
            raws.append((aidx_v[sl], bidx_v[sl]))
        for g, (araw, braw) in enumerate(raws):
            for j in range(L):
                i = g * L + j
                ta = lax.shift_right_logical(araw[j], 3)
                tb = lax.shift_right_logical(braw[j], 3)
                copies.append(
                    pltpu.async_copy(a_hbm.at[ta], abuf.at[i], sema))
                copies.append(
                    pltpu.async_copy(b_hbm.at[tb], bbuf.at[i], semb))
        return coff, copies, raws

    def drain_compute(coff, copies, raws, abuf, bbuf):
        for c in copies:
            c.wait()
        for g, (araw, braw) in enumerate(raws):
            acc = jnp.zeros((L,), jnp.float32)
            for j in range(L):
                i = g * L + j
                sa = lax.bitwise_and(araw[j], 7)
                sb = lax.bitwise_and(braw[j], 7)
                p = (abuf[i, sa, pl.ds(0, L)] * bbuf[i, sb, pl.ds(0, L)]
                     + abuf[i, sa, pl.ds(L, L)] * bbuf[i, sb, pl.ds(L, L)])
                acc = jnp.where(iota == j, jnp.sum(p), acc)
            out_v[pl.ds(coff + g * L, L)] = acc

    def chunk_pair(m, carry):
        st0 = half(m, 0, abuf, bbuf)
        st1 = half(m, 1, abuf1, bbuf1)
        drain_compute(*st0, abuf, bbuf)
        drain_compute(*st1, abuf1, bbuf1)
        return carry

    lax.fori_loop(0, NCH // 2, chunk_pair, 0)

    pltpu.sync_copy(out_v, out_hbm.at[pl.ds(base, BPW)])


def kernel(aIdx, bIdx, A, B):
    num = A.shape[0]
    k = pl.kernel(
        _sc_body,
        out_type=jax.ShapeDtypeStruct((BATCH,), jnp.float32),
        mesh=plsc.VectorSubcoreMesh(core_axis_name="c", subcore_axis_name="s"),
        compiler_params=pltpu.CompilerParams(needs_layout_passes=False),
        scratch_types=[
            pltpu.VMEM((BPW,), jnp.int32),
            pltpu.VMEM((BPW,), jnp.int32),
            pltpu.VMEM((CH, SUB, DIM), jnp.float32),
            pltpu.VMEM((CH, SUB, DIM), jnp.float32),
            pltpu.VMEM((CH, SUB, DIM), jnp.float32),
            pltpu.VMEM((CH, SUB, DIM), jnp.float32),
            pltpu.VMEM((BPW,), jnp.float32),
            pltpu.SemaphoreType.DMA,
            pltpu.SemaphoreType.DMA,
        ],
    )
    a3 = A.reshape(num // SUB, SUB, DIM)
    b3 = B.reshape(num // SUB, SUB, DIM)
    return k(aIdx.astype(jnp.int32), bIdx.astype(jnp.int32), a3, b3)
